# baseline (speedup 1.0000x reference)
import jax
import jax.numpy as jnp
from jax.experimental import pallas as pl

B, S, D = 16, 4096, 512
BLK = 1024


def _copy_block(x_ref, o_ref):
    o_ref[...] = x_ref[0]


def kernel(inputs):
    b, s = inputs.shape[0], inputs.shape[1]
    d = inputs.shape[2]
    nblk = s // BLK
    flat_values = pl.pallas_call(
        _copy_block,
        grid=(b, nblk),
        in_specs=[pl.BlockSpec((1, BLK, d), lambda i, j: (i, j, 0))],
        out_specs=pl.BlockSpec((BLK, d), lambda i, j: (i * (s // BLK) + j, 0)),
        out_shape=jax.ShapeDtypeStruct((b * s, d), inputs.dtype),
    )(inputs)
    row_splits = jnp.arange(0, b * s + 1, s, dtype=jnp.int64)
    return (flat_values, row_splits)


# speedup vs baseline: 1.1055x; 1.1055x over previous
import jax
import jax.numpy as jnp
from jax.experimental import pallas as pl
from jax.experimental.pallas import tpu as pltpu

B, S, D = 16, 4096, 512
BLK = 4096


def _copy_block(x_ref, o_ref):
    o_ref[...] = x_ref[0]


def kernel(inputs):
    b, s = inputs.shape[0], inputs.shape[1]
    d = inputs.shape[2]
    flat_values = pl.pallas_call(
        _copy_block,
        grid=(b,),
        in_specs=[pl.BlockSpec((1, BLK, d), lambda i: (i, 0, 0))],
        out_specs=pl.BlockSpec((BLK, d), lambda i: (i, 0)),
        out_shape=jax.ShapeDtypeStruct((b * s, d), inputs.dtype),
        compiler_params=pltpu.CompilerParams(
            dimension_semantics=("parallel",),
        ),
    )(inputs)
    row_splits = jnp.arange(0, b * s + 1, s, dtype=jnp.int64)
    return (flat_values, row_splits)
